# Initial kernel scaffold; baseline (speedup 1.0000x reference)
import jax
import jax.numpy as jnp
from jax.experimental import pallas as pl


def kernel(x, edge_index, W1, b1, W2, b2):
    raise NotImplementedError("write your pallas kernel here")


# speedup vs baseline: 27.0473x; 27.0473x over previous
import jax
import jax.numpy as jnp
from jax import lax
from jax.experimental import pallas as pl
from jax.experimental.pallas import tpu as pltpu
from jax.experimental.pallas import tpu_sc as plsc

N = 10000
E = 320000
D = 128
H = 16
C = 64

NC, NS = 2, 16
NW = NC * NS
EPW = E // NW
NB, B = 100, 100
NP = 10240
RPT = NP // NS

_MESH = plsc.VectorSubcoreMesh(
    core_axis_name="c", subcore_axis_name="s", num_cores=NC, num_subcores=NS)


def _deg_body(cols_ref, z_ref, ones_ref, out_ref, colv, zv, onesv, acc_sh):
  c = lax.axis_index("c")
  s = lax.axis_index("s")
  w = s * NC + c
  pltpu.sync_copy(cols_ref.at[w], colv)
  pltpu.sync_copy(z_ref, zv)
  pltpu.sync_copy(ones_ref, onesv)
  pltpu.sync_copy(zv, acc_sh.at[pl.ds(s * RPT, RPT)])
  plsc.subcore_barrier()

  def step(j, carry):
    pltpu.sync_copy(onesv, acc_sh.at[colv.at[j]], add=True)
    return carry

  lax.fori_loop(0, NB, step, 0)
  plsc.subcore_barrier()
  pltpu.sync_copy(acc_sh.at[pl.ds(s * RPT, RPT)],
                  out_ref.at[c, pl.ds(s * RPT, RPT)])


_SC_PARAMS = pltpu.CompilerParams(use_tc_tiling_on_sc=False)

_deg_call = pl.kernel(
    _deg_body,
    out_type=jax.ShapeDtypeStruct((NC, NP), jnp.float32),
    mesh=_MESH,
    compiler_params=_SC_PARAMS,
    scratch_types=[
        pltpu.VMEM((NB, B), jnp.int32),
        pltpu.VMEM((RPT,), jnp.float32),
        pltpu.VMEM((B,), jnp.float32),
        pltpu.VMEM_SHARED((NP,), jnp.float32),
    ],
)


def _make_agg(F):
  def body(y_ref, rows_ref, cols_ref, z_ref, out_ref, rowv, colv, msg, zv,
           acc_sh):
    c = lax.axis_index("c")
    s = lax.axis_index("s")
    w = s * NC + c
    pltpu.sync_copy(rows_ref.at[w], rowv)
    pltpu.sync_copy(cols_ref.at[w], colv)
    pltpu.sync_copy(z_ref, zv)
    pltpu.sync_copy(zv, acc_sh.at[pl.ds(s * RPT, RPT)])
    plsc.subcore_barrier()

    def step(j, carry):
      pltpu.sync_copy(y_ref.at[rowv.at[j]], msg)
      pltpu.sync_copy(msg, acc_sh.at[colv.at[j]], add=True)
      return carry

    lax.fori_loop(0, NB, step, 0)
    plsc.subcore_barrier()
    pltpu.sync_copy(acc_sh.at[pl.ds(s * RPT, RPT)],
                    out_ref.at[c, pl.ds(s * RPT, RPT)])

  return pl.kernel(
      body,
      out_type=jax.ShapeDtypeStruct((NC, NP, F), jnp.float32),
      mesh=_MESH,
      compiler_params=_SC_PARAMS,
      scratch_types=[
          pltpu.VMEM((NB, B), jnp.int32),
          pltpu.VMEM((NB, B), jnp.int32),
          pltpu.VMEM((B, F), jnp.float32),
          pltpu.VMEM((RPT, F), jnp.float32),
          pltpu.VMEM_SHARED((NP, F), jnp.float32),
      ],
  )


_agg1_call = _make_agg(H)
_agg2_call = _make_agg(C)


def _tc1_body(x_ref, w1_ref, d0_ref, d1_ref, y1_ref, dinv_ref):
  dinv = lax.rsqrt(d0_ref[...] + d1_ref[...] + 1.0)
  xw = jnp.dot(x_ref[...], w1_ref[...], preferred_element_type=jnp.float32)
  y1_ref[...] = xw * dinv
  dinv_ref[...] = dinv


def _tc2_body(a0_ref, a1_ref, y1_ref, dinv_ref, b1_ref, w2_ref, y2_ref):
  acc = a0_ref[...] + a1_ref[...] + y1_ref[...]
  h = jnp.maximum(acc * dinv_ref[...] + b1_ref[...], 0.0)
  y2_ref[...] = jnp.dot(
      h, w2_ref[...], preferred_element_type=jnp.float32) * dinv_ref[...]


def _tc3_body(a0_ref, a1_ref, y2_ref, dinv_ref, b2_ref, out_ref):
  o = (a0_ref[...] + a1_ref[...] + y2_ref[...]) * dinv_ref[...] + b2_ref[...]
  m = jnp.max(o, axis=1, keepdims=True)
  lse = jnp.log(jnp.sum(jnp.exp(o - m), axis=1, keepdims=True)) + m
  out_ref[...] = o - lse


def kernel(x, edge_index, W1, b1, W2, b2):
  rows = edge_index[0].reshape(NW, NB, B)
  cols = edge_index[1].reshape(NW, NB, B)
  z1 = jnp.zeros((RPT,), jnp.float32)
  zH = jnp.zeros((RPT, H), jnp.float32)
  zC = jnp.zeros((RPT, C), jnp.float32)
  ones_b = jnp.ones((B,), jnp.float32)

  degp = _deg_call(cols, z1, ones_b)
  d0 = degp[0, :N, None]
  d1 = degp[1, :N, None]

  y1, dinv = pl.pallas_call(
      _tc1_body,
      out_shape=(jax.ShapeDtypeStruct((N, H), jnp.float32),
                 jax.ShapeDtypeStruct((N, 1), jnp.float32)),
  )(x, W1, d0, d1)

  ap1 = _agg1_call(y1, rows, cols, zH)

  y2 = pl.pallas_call(
      _tc2_body,
      out_shape=jax.ShapeDtypeStruct((N, C), jnp.float32),
  )(ap1[0, :N], ap1[1, :N], y1, dinv, b1.reshape(1, H), W2)

  ap2 = _agg2_call(y2, rows, cols, zC)

  out = pl.pallas_call(
      _tc3_body,
      out_shape=jax.ShapeDtypeStruct((N, C), jnp.float32),
  )(ap2[0, :N], ap2[1, :N], y2, dinv, b2.reshape(1, C))
  return out


# speedup vs baseline: 39.0999x; 1.4456x over previous
import jax
import jax.numpy as jnp
from jax import lax
from jax.experimental import pallas as pl
from jax.experimental.pallas import tpu as pltpu
from jax.experimental.pallas import tpu_sc as plsc

N = 10000
E = 320000
D = 128
H = 16
C = 64

NC, NS = 2, 16
NW = NC * NS
EPW = E // NW
NB, B = 100, 100
NP = 10240
RPT = NP // NS

_MESH = plsc.VectorSubcoreMesh(
    core_axis_name="c", subcore_axis_name="s", num_cores=NC, num_subcores=NS)


def _deg_body(cols_ref, z_ref, ones_ref, out_ref, colv, zv, onesv, acc_sh):
  c = lax.axis_index("c")
  s = lax.axis_index("s")
  w = s * NC + c
  pltpu.sync_copy(cols_ref.at[w], colv)
  pltpu.sync_copy(z_ref, zv)
  pltpu.sync_copy(ones_ref, onesv)
  pltpu.sync_copy(zv, acc_sh.at[pl.ds(s * RPT, RPT)])
  plsc.subcore_barrier()

  def step(j, carry):
    pltpu.sync_copy(onesv, acc_sh.at[colv.at[j]], add=True)
    return carry

  lax.fori_loop(0, NB, step, 0)
  plsc.subcore_barrier()
  pltpu.sync_copy(acc_sh.at[pl.ds(s * RPT, RPT)],
                  out_ref.at[c, pl.ds(s * RPT, RPT)])


_SC_PARAMS = pltpu.CompilerParams(use_tc_tiling_on_sc=False)

_deg_call = pl.kernel(
    _deg_body,
    out_type=jax.ShapeDtypeStruct((NC, NP), jnp.float32),
    mesh=_MESH,
    compiler_params=_SC_PARAMS,
    scratch_types=[
        pltpu.VMEM((NB, B), jnp.int32),
        pltpu.VMEM((RPT,), jnp.float32),
        pltpu.VMEM((B,), jnp.float32),
        pltpu.VMEM_SHARED((NP,), jnp.float32),
    ],
)


K = 10


def _make_agg(F):
  def body(y_ref, rows_ref, cols_ref, z_ref, out_ref, rowv, colv, msg,
           gsem, ssem, acc_sh):
    c = lax.axis_index("c")
    s = lax.axis_index("s")
    w = s * NC + c
    pltpu.sync_copy(rows_ref.at[w], rowv)
    pltpu.sync_copy(cols_ref.at[w], colv)
    pltpu.sync_copy(z_ref, msg.at[pl.ds(0, RPT)])
    pltpu.sync_copy(msg.at[pl.ds(0, RPT)], acc_sh.at[pl.ds(s * RPT, RPT)])
    plsc.subcore_barrier()

    def group(g, carry):
      j0 = g * K
      gathers = [
          pltpu.async_copy(y_ref.at[rowv.at[j0 + b]],
                           msg.at[pl.ds(b * B, B)], gsem)
          for b in range(K)
      ]
      for d in gathers:
        d.wait()
      scatters = [
          pltpu.async_copy(msg.at[pl.ds(b * B, B)],
                           acc_sh.at[colv.at[j0 + b]], ssem, add=True)
          for b in range(K)
      ]
      for d in scatters:
        d.wait()
      return carry

    lax.fori_loop(0, NB // K, group, 0)
    plsc.subcore_barrier()
    pltpu.sync_copy(acc_sh.at[pl.ds(s * RPT, RPT)],
                    out_ref.at[c, pl.ds(s * RPT, RPT)])

  return pl.kernel(
      body,
      out_type=jax.ShapeDtypeStruct((NC, NP, F), jnp.float32),
      mesh=_MESH,
      compiler_params=_SC_PARAMS,
      scratch_types=[
          pltpu.VMEM((NB, B), jnp.int32),
          pltpu.VMEM((NB, B), jnp.int32),
          pltpu.VMEM((K * B, F), jnp.float32),
          pltpu.SemaphoreType.DMA,
          pltpu.SemaphoreType.DMA,
          pltpu.VMEM_SHARED((NP, F), jnp.float32),
      ],
  )


_agg1_call = _make_agg(H)
_agg2_call = _make_agg(C)


def _tc1_body(x_ref, w1_ref, d0_ref, d1_ref, y1_ref, dinv_ref):
  dinv = lax.rsqrt(d0_ref[...] + d1_ref[...] + 1.0)
  xw = jnp.dot(x_ref[...], w1_ref[...], preferred_element_type=jnp.float32)
  y1_ref[...] = xw * dinv
  dinv_ref[...] = dinv


def _tc2_body(a0_ref, a1_ref, y1_ref, dinv_ref, b1_ref, w2_ref, y2_ref):
  acc = a0_ref[...] + a1_ref[...] + y1_ref[...]
  h = jnp.maximum(acc * dinv_ref[...] + b1_ref[...], 0.0)
  y2_ref[...] = jnp.dot(
      h, w2_ref[...], preferred_element_type=jnp.float32) * dinv_ref[...]


def _tc3_body(a0_ref, a1_ref, y2_ref, dinv_ref, b2_ref, out_ref):
  o = (a0_ref[...] + a1_ref[...] + y2_ref[...]) * dinv_ref[...] + b2_ref[...]
  m = jnp.max(o, axis=1, keepdims=True)
  lse = jnp.log(jnp.sum(jnp.exp(o - m), axis=1, keepdims=True)) + m
  out_ref[...] = o - lse


def kernel(x, edge_index, W1, b1, W2, b2):
  rows = edge_index[0].reshape(NW, NB, B)
  cols = edge_index[1].reshape(NW, NB, B)
  z1 = jnp.zeros((RPT,), jnp.float32)
  zH = jnp.zeros((RPT, H), jnp.float32)
  zC = jnp.zeros((RPT, C), jnp.float32)
  ones_b = jnp.ones((B,), jnp.float32)

  degp = _deg_call(cols, z1, ones_b)
  d0 = degp[0, :N, None]
  d1 = degp[1, :N, None]

  y1, dinv = pl.pallas_call(
      _tc1_body,
      out_shape=(jax.ShapeDtypeStruct((N, H), jnp.float32),
                 jax.ShapeDtypeStruct((N, 1), jnp.float32)),
  )(x, W1, d0, d1)

  ap1 = _agg1_call(y1, rows, cols, zH)

  y2 = pl.pallas_call(
      _tc2_body,
      out_shape=jax.ShapeDtypeStruct((N, C), jnp.float32),
  )(ap1[0, :N], ap1[1, :N], y1, dinv, b1.reshape(1, H), W2)

  ap2 = _agg2_call(y2, rows, cols, zC)

  out = pl.pallas_call(
      _tc3_body,
      out_shape=jax.ShapeDtypeStruct((N, C), jnp.float32),
  )(ap2[0, :N], ap2[1, :N], y2, dinv, b2.reshape(1, C))
  return out


# speedup vs baseline: 41.1003x; 1.0512x over previous
import jax
import jax.numpy as jnp
from jax import lax
from jax.experimental import pallas as pl
from jax.experimental.pallas import tpu as pltpu
from jax.experimental.pallas import tpu_sc as plsc

N = 10000
E = 320000
D = 128
H = 16
C = 64

NC, NS = 2, 16
NW = NC * NS
EPW = E // NW
NB, B = 100, 100
NP = 10240
RPT = NP // NS
K = 10

_MESH = plsc.VectorSubcoreMesh(
    core_axis_name="c", subcore_axis_name="s", num_cores=NC, num_subcores=NS)
_SC_PARAMS = pltpu.CompilerParams(use_tc_tiling_on_sc=False)


def _deg_body(cols_ref, z_ref, ones_ref, out_ref, colv, zv, onesv, acc_sh):
  c = lax.axis_index("c")
  s = lax.axis_index("s")
  w = s * NC + c
  pltpu.sync_copy(cols_ref.at[w], colv)
  pltpu.sync_copy(z_ref, zv)
  pltpu.sync_copy(ones_ref, onesv)
  pltpu.sync_copy(zv, acc_sh.at[pl.ds(s * RPT, RPT)])
  plsc.subcore_barrier()

  def step(j, carry):
    pltpu.sync_copy(onesv, acc_sh.at[colv.at[j]], add=True)
    return carry

  lax.fori_loop(0, NB, step, 0)
  plsc.subcore_barrier()
  pltpu.sync_copy(acc_sh.at[pl.ds(s * RPT, RPT)],
                  out_ref.at[c, pl.ds(s * RPT, RPT)])


_deg_call = pl.kernel(
    _deg_body,
    out_type=jax.ShapeDtypeStruct((NC, NP, 1), jnp.float32),
    mesh=_MESH,
    compiler_params=_SC_PARAMS,
    scratch_types=[
        pltpu.VMEM((NB, B), jnp.int32),
        pltpu.VMEM((RPT, 1), jnp.float32),
        pltpu.VMEM((B, 1), jnp.float32),
        pltpu.VMEM_SHARED((NP, 1), jnp.float32),
    ],
)


def _make_agg(F):
  def body(y_ref, rows_ref, cols_ref, z_ref, out_ref, rowv, colv, msg,
           gsem, ssem, acc_sh):
    c = lax.axis_index("c")
    s = lax.axis_index("s")
    w = s * NC + c
    pltpu.sync_copy(rows_ref.at[w], rowv)
    pltpu.sync_copy(cols_ref.at[w], colv)
    pltpu.sync_copy(z_ref, msg.at[pl.ds(0, RPT)])
    pltpu.sync_copy(msg.at[pl.ds(0, RPT)], acc_sh.at[pl.ds(s * RPT, RPT)])
    plsc.subcore_barrier()

    def group(g, carry):
      j0 = g * K
      gathers = [
          pltpu.async_copy(y_ref.at[rowv.at[j0 + b]],
                           msg.at[pl.ds(b * B, B)], gsem)
          for b in range(K)
      ]
      for d in gathers:
        d.wait()
      scatters = [
          pltpu.async_copy(msg.at[pl.ds(b * B, B)],
                           acc_sh.at[colv.at[j0 + b]], ssem, add=True)
          for b in range(K)
      ]
      for d in scatters:
        d.wait()
      return carry

    lax.fori_loop(0, NB // K, group, 0)
    plsc.subcore_barrier()
    pltpu.sync_copy(acc_sh.at[pl.ds(s * RPT, RPT)],
                    out_ref.at[c, pl.ds(s * RPT, RPT)])

  return pl.kernel(
      body,
      out_type=jax.ShapeDtypeStruct((NC, NP, F), jnp.float32),
      mesh=_MESH,
      compiler_params=_SC_PARAMS,
      scratch_types=[
          pltpu.VMEM((NB, B), jnp.int32),
          pltpu.VMEM((NB, B), jnp.int32),
          pltpu.VMEM((K * B, F), jnp.float32),
          pltpu.SemaphoreType.DMA,
          pltpu.SemaphoreType.DMA,
          pltpu.VMEM_SHARED((NP, F), jnp.float32),
      ],
  )


_agg1_call = _make_agg(H)
_agg2_call = _make_agg(C)


def _tc0_body(x_ref, w1_ref, xw_ref):
  xw_ref[...] = jnp.dot(x_ref[...], w1_ref[...],
                        preferred_element_type=jnp.float32)


def _tc1_body(xw_ref, degp_ref, y1_ref, dinv_ref):
  deg = degp_ref[0][:N] + degp_ref[1][:N] + 1.0
  dinv = lax.rsqrt(deg)
  y1_ref[...] = xw_ref[...] * dinv
  dinv_ref[...] = dinv


def _tc2_body(ap_ref, y1_ref, dinv_ref, b1_ref, w2_ref, y2_ref):
  acc = ap_ref[0][:N] + ap_ref[1][:N] + y1_ref[...]
  h = jnp.maximum(acc * dinv_ref[...] + b1_ref[...], 0.0)
  y2_ref[...] = jnp.dot(
      h, w2_ref[...], preferred_element_type=jnp.float32) * dinv_ref[...]


def _tc3_body(ap_ref, y2_ref, dinv_ref, b2_ref, out_ref):
  o = (ap_ref[0][:N] + ap_ref[1][:N] + y2_ref[...]) * dinv_ref[...]
  o = o + b2_ref[...]
  m = jnp.max(o, axis=1, keepdims=True)
  lse = jnp.log(jnp.sum(jnp.exp(o - m), axis=1, keepdims=True)) + m
  out_ref[...] = o - lse


def kernel(x, edge_index, W1, b1, W2, b2):
  rows = edge_index[0].reshape(NW, NB, B)
  cols = edge_index[1].reshape(NW, NB, B)
  z1 = jnp.zeros((RPT, 1), jnp.float32)
  zH = jnp.zeros((RPT, H), jnp.float32)
  zC = jnp.zeros((RPT, C), jnp.float32)
  ones_b = jnp.ones((B, 1), jnp.float32)

  xw = pl.pallas_call(
      _tc0_body,
      out_shape=jax.ShapeDtypeStruct((N, H), jnp.float32),
  )(x, W1)

  degp = _deg_call(cols, z1, ones_b)

  y1, dinv = pl.pallas_call(
      _tc1_body,
      out_shape=(jax.ShapeDtypeStruct((N, H), jnp.float32),
                 jax.ShapeDtypeStruct((N, 1), jnp.float32)),
  )(xw, degp)

  ap1 = _agg1_call(y1, rows, cols, zH)

  y2 = pl.pallas_call(
      _tc2_body,
      out_shape=jax.ShapeDtypeStruct((N, C), jnp.float32),
  )(ap1, y1, dinv, b1.reshape(1, H), W2)

  ap2 = _agg2_call(y2, rows, cols, zC)

  out = pl.pallas_call(
      _tc3_body,
      out_shape=jax.ShapeDtypeStruct((N, C), jnp.float32),
  )(ap2, y2, dinv, b2.reshape(1, C))
  return out
